# Initial kernel scaffold; baseline (speedup 1.0000x reference)
import jax
import jax.numpy as jnp
from jax.experimental import pallas as pl


def kernel(features, edge_index, W1, b1, Wc1, bc1, Wc2, bc2, W2, b2):
    raise NotImplementedError("write your pallas kernel here")


# speedup vs baseline: 5.7992x; 5.7992x over previous
import functools

import jax
import jax.numpy as jnp
from jax import lax
from jax.experimental import pallas as pl
from jax.experimental.pallas import tpu as pltpu
from jax.experimental.pallas import tpu_sc as plsc

N = 10000
E = 320000
NSUB = 16
CH = 128
F32 = jnp.float32

NSLAB = 79
ACC_ROWS = NSLAB * CH
NFULL = N // CH
TAIL0 = NFULL * CH
TAIL = N - TAIL0

NHALF = N // 2
ACC2_ROWS = 5120
SCRAP2 = 5112
WFULL = NHALF // CH
WTAIL0 = WFULL * CH
WTAIL = NHALF - WTAIL0

CHUNKS16 = (E // NSUB + CH - 1) // CH
CHUNKS32 = (E // (2 * NSUB) + CH - 1) // CH
GROUPS16 = CHUNKS16 * (CH // 16)

_mesh = plsc.VectorSubcoreMesh(core_axis_name="c", subcore_axis_name="s")


DEGW = 128


@functools.partial(
    pl.kernel,
    mesh=_mesh,
    out_type=(
        jax.ShapeDtypeStruct((N, DEGW), F32),
        jax.ShapeDtypeStruct((N, DEGW), F32),
    ),
    scratch_types=[
        pltpu.VMEM((CHUNKS32, CH), jnp.int32),
        pltpu.VMEM((CH, DEGW), F32),
        pltpu.VMEM_SHARED((ACC_ROWS, DEGW), F32),
    ],
)
def _deg_kernel(dstd, ones_h, z_h, dlo, dhi, dst_v, ones_v, acc):
    cid = lax.axis_index("c")
    sid = lax.axis_index("s")
    tid = cid * NSUB + sid
    pltpu.sync_copy(dstd.at[tid], dst_v)
    pltpu.sync_copy(ones_h, ones_v)
    for k in range(5):
        c = sid * 5 + k
        rof = pl.multiple_of(c * CH, CH)

        @pl.when(c < NSLAB)
        def _():
            pltpu.sync_copy(z_h, acc.at[pl.ds(rof, CH)])

    plsc.subcore_barrier()

    def body(i, carry):
        pltpu.sync_copy(ones_v, acc.at[dst_v.at[i]], add=True)
        return carry

    lax.fori_loop(0, CHUNKS32, body, 0)
    plsc.subcore_barrier()
    for k in range(5):
        c = sid * 5 + k
        rof = pl.multiple_of(c * CH, CH)

        @pl.when((c < NFULL) & (cid == 0))
        def _():
            pltpu.sync_copy(acc.at[pl.ds(rof, CH)], dlo.at[pl.ds(rof, CH)])

        @pl.when((c < NFULL) & (cid == 1))
        def _():
            pltpu.sync_copy(acc.at[pl.ds(rof, CH)], dhi.at[pl.ds(rof, CH)])

    @pl.when((sid == NSUB - 1) & (cid == 0))
    def _():
        pltpu.sync_copy(acc.at[pl.ds(TAIL0, TAIL)], dlo.at[pl.ds(TAIL0, TAIL)])

    @pl.when((sid == NSUB - 1) & (cid == 1))
    def _():
        pltpu.sync_copy(acc.at[pl.ds(TAIL0, TAIL)], dhi.at[pl.ds(TAIL0, TAIL)])


@functools.partial(
    pl.kernel,
    mesh=_mesh,
    out_type=jax.ShapeDtypeStruct((N, 128), F32),
    scratch_types=[
        pltpu.VMEM((CHUNKS16, CH), jnp.int32),
        pltpu.VMEM((CHUNKS16, CH), jnp.int32),
        pltpu.VMEM((CH, 128), F32),
        pltpu.VMEM_SHARED((ACC2_ROWS, 128), F32),
        pltpu.SemaphoreType.DMA,
    ],
)
def _spmm(g, srcp, dstp, zrows, out, src_v, dst_v, rows_v, acc, sem):
    cid = lax.axis_index("c")
    sid = lax.axis_index("s")
    base = cid * NHALF
    pltpu.sync_copy(srcp.at[sid], src_v)
    pltpu.sync_copy(dstp.at[sid], dst_v)

    for k in range(3):
        c = sid * 3 + k
        rof = pl.multiple_of(c * CH, CH)

        @pl.when(c < ACC2_ROWS // CH)
        def _():
            pltpu.sync_copy(zrows, acc.at[pl.ds(rof, CH)])

    def tbody(gidx, carry):
        r = gidx // (CH // 16)
        c0 = (gidx % (CH // 16)) * 16
        d = dst_v[r, pl.ds(c0, 16)] - base
        ok = (d >= 0) & (d < NHALF)
        dst_v[r, pl.ds(c0, 16)] = jnp.where(ok, d, SCRAP2)
        return carry

    lax.fori_loop(0, GROUPS16, tbody, 0)
    plsc.subcore_barrier()

    def body(i, carry):
        pltpu.async_copy(g.at[src_v.at[i]], rows_v, sem).wait()
        pltpu.sync_copy(rows_v, acc.at[dst_v.at[i]], add=True)
        return carry

    lax.fori_loop(0, CHUNKS16, body, 0)
    plsc.subcore_barrier()

    for k in range(3):
        c = sid * 3 + k
        rof = pl.multiple_of(c * CH, CH)
        oof = pl.multiple_of(base + c * CH, 8)

        @pl.when(c < WFULL)
        def _():
            pltpu.sync_copy(acc.at[pl.ds(rof, CH)], out.at[pl.ds(oof, CH)])

    @pl.when(sid == NSUB - 1)
    def _():
        oof = pl.multiple_of(base + WTAIL0, 8)
        pltpu.sync_copy(acc.at[pl.ds(WTAIL0, WTAIL)], out.at[pl.ds(oof, WTAIL)])


RB = 1000
GRID = N // RB


def _rb(c):
    return pl.BlockSpec((RB, c), lambda i: (i, 0))


def _wfull(r, c):
    return pl.BlockSpec((r, c), lambda i: (0, 0))


def _dot(a, b):
    return jnp.dot(a, b, preferred_element_type=F32)


def _scale(dlo_ref, dhi_ref):
    return lax.rsqrt(dlo_ref[:, 0:1] + dhi_ref[:, 0:1] + 1.0)


def _tc1_body(f_ref, w1_ref, b1_ref, wc1_ref, dlo_ref, dhi_ref,
              x_ref, g_ref):
    xb = jnp.maximum(_dot(f_ref[...], w1_ref[...]) + b1_ref[...], 0.0)
    x_ref[...] = xb
    s = _scale(dlo_ref, dhi_ref)
    g_ref[...] = s * _dot(xb, wc1_ref[...])


_tc1 = pl.pallas_call(
    _tc1_body,
    grid=(GRID,),
    in_specs=[_rb(128), _wfull(128, 128), _wfull(1, 128), _wfull(128, 128),
              _rb(DEGW), _rb(DEGW)],
    out_specs=[_rb(128), _rb(128)],
    out_shape=[jax.ShapeDtypeStruct((N, 128), F32),
               jax.ShapeDtypeStruct((N, 128), F32)],
)


def _tc2_body(a_ref, g_in, dlo, dhi, b_ref, w_ref, x_ref, g_ref):
    s = _scale(dlo, dhi)
    xn = s * (a_ref[...] + g_in[...]) + b_ref[...]
    x_ref[...] = xn
    g_ref[...] = s * _dot(xn, w_ref[...])


_tc2 = pl.pallas_call(
    _tc2_body,
    grid=(GRID,),
    in_specs=[_rb(128), _rb(128), _rb(DEGW), _rb(DEGW),
              _wfull(1, 128), _wfull(128, 128)],
    out_specs=[_rb(128), _rb(128)],
    out_shape=[jax.ShapeDtypeStruct((N, 128), F32),
               jax.ShapeDtypeStruct((N, 128), F32)],
)


def _tc3_body(a_ref, g_in, dlo, dhi, bc1_ref, x11_ref, wc2_ref,
              x12_ref, nlo, nhi):
    s = _scale(dlo, dhi)
    x12 = s * (a_ref[...] + g_in[...]) + bc1_ref[...]
    x12_ref[...] = x12
    w = wc2_ref[...]
    g3 = s * (_dot(x11_ref[...], w[0:128, :]) + _dot(x12, w[128:256, :]))
    nlo[...] = g3[:, :128]
    nhi[...] = g3[:, 128:]


_tc3 = pl.pallas_call(
    _tc3_body,
    grid=(GRID,),
    in_specs=[_rb(128), _rb(128), _rb(DEGW), _rb(DEGW),
              _wfull(1, 128), _rb(128), _wfull(256, 256)],
    out_specs=[_rb(128), _rb(128), _rb(128)],
    out_shape=[jax.ShapeDtypeStruct((N, 128), F32),
               jax.ShapeDtypeStruct((N, 128), F32),
               jax.ShapeDtypeStruct((N, 128), F32)],
)


def _tc4_body(alo, ahi, glo, ghi, dlo, dhi, bc2_ref, wc2_ref,
              x21_ref, nlo, nhi):
    s = _scale(dlo, dhi)
    x21 = s * jnp.concatenate(
        [alo[...] + glo[...], ahi[...] + ghi[...]], axis=1) + bc2_ref[...]
    x21_ref[...] = x21
    g4 = s * _dot(x21, wc2_ref[...])
    nlo[...] = g4[:, :128]
    nhi[...] = g4[:, 128:]


_tc4 = pl.pallas_call(
    _tc4_body,
    grid=(GRID,),
    in_specs=[_rb(128), _rb(128), _rb(128), _rb(128), _rb(DEGW), _rb(DEGW),
              _wfull(1, 256), _wfull(256, 256)],
    out_specs=[_rb(256), _rb(128), _rb(128)],
    out_shape=[jax.ShapeDtypeStruct((N, 256), F32),
               jax.ShapeDtypeStruct((N, 128), F32),
               jax.ShapeDtypeStruct((N, 128), F32)],
)


def _tc5_body(alo, ahi, glo, ghi, dlo, dhi, bc2_ref,
              x_ref, x11_ref, x12_ref, x21_ref, w2_ref, b2_ref, o_ref):
    s = _scale(dlo, dhi)
    x22 = s * jnp.concatenate(
        [alo[...] + glo[...], ahi[...] + ghi[...]], axis=1) + bc2_ref[...]
    w2 = w2_ref[...]
    o_ref[...] = (_dot(x_ref[...], w2[0:128, :])
                  + _dot(x11_ref[...], w2[128:256, :])
                  + _dot(x12_ref[...], w2[256:384, :])
                  + _dot(x21_ref[...], w2[384:640, :])
                  + _dot(x22, w2[640:896, :])
                  + b2_ref[...])


_tc5 = pl.pallas_call(
    _tc5_body,
    grid=(GRID,),
    in_specs=[_rb(128), _rb(128), _rb(128), _rb(128), _rb(DEGW), _rb(DEGW),
              _wfull(1, 256), _rb(128), _rb(128), _rb(128), _rb(256),
              _wfull(896, 64), _wfull(1, 64)],
    out_specs=[_rb(64)],
    out_shape=[jax.ShapeDtypeStruct((N, 64), F32)],
)


def kernel(features, edge_index, W1, b1, Wc1, bc1, Wc2, bc2, W2, b2):
    src = edge_index[0]
    dst = edge_index[1]
    e32 = E // (2 * NSUB)
    e16 = E // NSUB
    dstp32 = jnp.pad(dst.reshape(2 * NSUB, e32),
                     ((0, 0), (0, CHUNKS32 * CH - e32)),
                     constant_values=N).reshape(2 * NSUB, CHUNKS32, CH)
    srcp16 = jnp.pad(src.reshape(NSUB, e16),
                     ((0, 0), (0, CHUNKS16 * CH - e16)),
                     constant_values=0).reshape(NSUB, CHUNKS16, CH)
    dstp16 = jnp.pad(dst.reshape(NSUB, e16),
                     ((0, 0), (0, CHUNKS16 * CH - e16)),
                     constant_values=N).reshape(NSUB, CHUNKS16, CH)
    ones_h = jnp.ones((CH, DEGW), F32)
    z128 = jnp.zeros((CH, 128), F32)
    b1r = b1.reshape(1, -1)
    bc1r = bc1.reshape(1, -1)
    bc2r = bc2.reshape(1, -1)
    b2r = b2.reshape(1, -1)

    dlo, dhi = _deg_kernel(dstp32, ones_h, z128)
    x, g1 = _tc1(features, W1, b1r, Wc1, dlo, dhi)
    a1 = _spmm(g1, srcp16, dstp16, z128)
    x11, g2 = _tc2(a1, g1, dlo, dhi, bc1r, Wc1)
    a2 = _spmm(g2, srcp16, dstp16, z128)
    x12, g3lo, g3hi = _tc3(a2, g2, dlo, dhi, bc1r, x11, Wc2)
    a3lo = _spmm(g3lo, srcp16, dstp16, z128)
    a3hi = _spmm(g3hi, srcp16, dstp16, z128)
    x21, g4lo, g4hi = _tc4(a3lo, a3hi, g3lo, g3hi, dlo, dhi, bc2r, Wc2)
    a4lo = _spmm(g4lo, srcp16, dstp16, z128)
    a4hi = _spmm(g4hi, srcp16, dstp16, z128)
    (out,) = _tc5(a4lo, a4hi, g4lo, g4hi, dlo, dhi, bc2r,
                  x, x11, x12, x21, W2, b2r)
    return out
